# baseline (speedup 1.0000x reference)
import functools

import jax
import jax.numpy as jnp
from jax import lax
from jax.experimental import pallas as pl
from jax.experimental.pallas import tpu as pltpu
from jax.experimental.pallas import tpu_sc as plsc

NB = 16
NF = 80
NT = 8192
NS = 512
MAXD = 15
ROWS_PER_W = 40
CH = 5
NCHUNK = ROWS_PER_W // CH

_mesh = plsc.VectorSubcoreMesh(core_axis_name="c", subcore_axis_name="s")


@functools.partial(
    pl.kernel,
    mesh=_mesh,
    compiler_params=pltpu.CompilerParams(
        needs_layout_passes=False, use_tc_tiling_on_sc=False
    ),
    out_type=jax.ShapeDtypeStruct((NB, NF, NS), jnp.float32),
    scratch_types=[
        pltpu.VMEM((NS,), jnp.int32),
        pltpu.VMEM((NS,), jnp.int32),
        pltpu.VMEM((ROWS_PER_W, NS), jnp.float32),
        pltpu.VMEM((CH + 1, NT), jnp.float32),
        pltpu.VMEM((CH + 1, NT), jnp.float32),
        pltpu.SemaphoreType.DMA,
        pltpu.SemaphoreType.DMA,
    ],
)
def _seg_avg(x_hbm, dur_hbm, out_hbm, dur_v, starts_v, out_v, buf_a, buf_b,
             sem_a, sem_b):
    cid = lax.axis_index("c")
    sid = lax.axis_index("s")
    wid = sid * 2 + cid
    batch = wid // 2
    f0 = (wid % 2) * ROWS_PER_W

    pltpu.sync_copy(dur_hbm.at[batch], dur_v)

    iota = jnp.arange(16, dtype=jnp.int32)
    lane15 = jnp.full((16,), 15, jnp.int32)

    def bounds_body(g, carry_v):
        base = pl.multiple_of(g * 16, 16)
        d = dur_v[pl.ds(base, 16)]
        ends = d
        for k in (1, 2, 4, 8):
            sh = ends.at[jnp.maximum(iota - k, 0)].get(mode="promise_in_bounds")
            ends = ends + jnp.where(iota >= k, sh, 0)
        ends = ends + carry_v
        starts_v[pl.ds(base, 16)] = ends - d
        return ends.at[lane15].get(mode="promise_in_bounds")

    lax.fori_loop(0, NS // 16, bounds_body, jnp.zeros((16,), jnp.int32))

    zeros = jnp.zeros((16,), jnp.float32)

    def zero_body(i, _):
        base = pl.multiple_of(i * 16, 16)
        buf_a[CH, pl.ds(base, 16)] = zeros
        buf_b[CH, pl.ds(base, 16)] = zeros
        return 0

    lax.fori_loop(0, NT // 16, zero_body, 0)

    guard = jnp.full((16,), CH, jnp.int32)

    def compute_row(buf, r_local, r):
        rvec = jnp.full((16,), 1, jnp.int32) * r_local

        def grp_body(g, _2):
            base = pl.multiple_of(g * 32, 16)
            s0 = starts_v[pl.ds(base, 16)]
            l0 = dur_v[pl.ds(base, 16)]
            s1 = starts_v[pl.ds(base + 16, 16)]
            l1 = dur_v[pl.ds(base + 16, 16)]
            acc0 = cnt0 = acc1 = cnt1 = zeros
            for d in range(MAXD):
                r0 = jnp.where(l0 > d, rvec, guard)
                r1 = jnp.where(l1 > d, rvec, guard)
                v0 = plsc.load_gather(buf, [r0, s0 + d])
                v1 = plsc.load_gather(buf, [r1, s1 + d])
                acc0 = acc0 + v0
                acc1 = acc1 + v1
                cnt0 = cnt0 + jnp.where(v0 == 0.0, 0.0, 1.0)
                cnt1 = cnt1 + jnp.where(v1 == 0.0, 0.0, 1.0)
            out_v[r, pl.ds(base, 16)] = acc0 / jnp.maximum(cnt0, 1.0)
            out_v[r, pl.ds(base + 16, 16)] = acc1 / jnp.maximum(cnt1, 1.0)
            return 0

        lax.fori_loop(0, NS // 32, grp_body, 0)

    def fire(buf, c, sem):
        pltpu.async_copy(
            x_hbm.at[batch, pl.ds(f0 + c * CH, CH)], buf.at[pl.ds(0, CH)], sem
        )

    def drain(buf, c, sem):
        pltpu.make_async_copy(
            x_hbm.at[batch, pl.ds(f0 + c * CH, CH)], buf.at[pl.ds(0, CH)], sem
        ).wait()

    def compute_chunk(buf, c):
        def row_body(j, _):
            compute_row(buf, j, c * CH + j)
            return 0

        lax.fori_loop(0, CH, row_body, 0)

    fire(buf_a, 0, sem_a)

    def pair_body(p, _):
        ca = 2 * p
        fire(buf_b, ca + 1, sem_b)
        drain(buf_a, ca, sem_a)
        compute_chunk(buf_a, ca)

        @pl.when(p < NCHUNK // 2 - 1)
        def _prefetch():
            fire(buf_a, ca + 2, sem_a)

        drain(buf_b, ca + 1, sem_b)
        compute_chunk(buf_b, ca + 1)
        return 0

    lax.fori_loop(0, NCHUNK // 2, pair_body, 0)
    pltpu.sync_copy(out_v, out_hbm.at[batch, pl.ds(f0, ROWS_PER_W)])


def kernel(x, durations):
    return _seg_avg(x, durations.astype(jnp.int32))


# speedup vs baseline: 1.6120x; 1.6120x over previous
import functools

import jax
import jax.numpy as jnp
from jax import lax
from jax.experimental import pallas as pl
from jax.experimental.pallas import tpu as pltpu
from jax.experimental.pallas import tpu_sc as plsc

NB = 16
NF = 80
NT = 8192
NS = 512
MAXD = 15
ROWS_PER_W = 40
RING = 10
DEAD = RING * NT

_mesh = plsc.VectorSubcoreMesh(core_axis_name="c", subcore_axis_name="s")


@functools.partial(
    pl.kernel,
    mesh=_mesh,
    compiler_params=pltpu.CompilerParams(needs_layout_passes=False),
    out_type=jax.ShapeDtypeStruct((NB, NF, NS), jnp.float32),
    scratch_types=[
        pltpu.VMEM((NS,), jnp.int32),
        pltpu.VMEM((NS,), jnp.int32),
        pltpu.VMEM((ROWS_PER_W, NS), jnp.float32),
        pltpu.VMEM((RING * NT + 16,), jnp.float32),
        pltpu.SemaphoreType.DMA,
    ],
)
def _seg_avg(x_hbm, dur_hbm, out_hbm, dur_v, starts_v, out_v, ring_v, sem):
    cid = lax.axis_index("c")
    sid = lax.axis_index("s")
    wid = sid * 2 + cid
    batch = wid // 2
    f0 = (wid % 2) * ROWS_PER_W

    pltpu.sync_copy(dur_hbm.at[batch], dur_v)

    iota = jnp.arange(16, dtype=jnp.int32)
    lane15 = jnp.full((16,), 15, jnp.int32)

    def bounds_body(g, carry_v):
        base = pl.multiple_of(g * 16, 16)
        d = dur_v[pl.ds(base, 16)]
        ends = d
        for k in (1, 2, 4, 8):
            sh = ends.at[jnp.maximum(iota - k, 0)].get(mode="promise_in_bounds")
            ends = ends + jnp.where(iota >= k, sh, 0)
        ends = ends + carry_v
        starts_v[pl.ds(base, 16)] = ends - d
        return ends.at[lane15].get(mode="promise_in_bounds")

    lax.fori_loop(0, NS // 16, bounds_body, jnp.zeros((16,), jnp.int32))

    zeros = jnp.zeros((16,), jnp.float32)
    ring_v[pl.ds(DEAD, 16)] = zeros

    def compute_row(qbase, r):
        def grp_body(g, _2):
            base = pl.multiple_of(g * 32, 16)
            s0 = starts_v[pl.ds(base, 16)] + qbase
            l0 = dur_v[pl.ds(base, 16)]
            s1 = starts_v[pl.ds(base + 16, 16)] + qbase
            l1 = dur_v[pl.ds(base + 16, 16)]
            acc0 = cnt0 = acc1 = cnt1 = zeros
            for d in range(MAXD):
                i0 = jnp.where(l0 > d, s0 + d, DEAD)
                i1 = jnp.where(l1 > d, s1 + d, DEAD)
                v0 = plsc.load_gather(ring_v, [i0])
                v1 = plsc.load_gather(ring_v, [i1])
                acc0 = acc0 + v0
                acc1 = acc1 + v1
                cnt0 = cnt0 + jnp.where(v0 == 0.0, 0.0, 1.0)
                cnt1 = cnt1 + jnp.where(v1 == 0.0, 0.0, 1.0)
            out_v[r, pl.ds(base, 16)] = acc0 / jnp.maximum(cnt0, 1.0)
            out_v[r, pl.ds(base + 16, 16)] = acc1 / jnp.maximum(cnt1, 1.0)
            return 0

        lax.fori_loop(0, NS // 32, grp_body, 0)

    for q in range(RING):
        pltpu.async_copy(
            x_hbm.at[batch, f0 + q], ring_v.at[pl.ds(q * NT, NT)], sem
        )

    def row_body(r, q):
        qbase = pl.multiple_of(q * NT, 16)
        pltpu.make_async_copy(
            x_hbm.at[batch, f0], ring_v.at[pl.ds(qbase, NT)], sem
        ).wait()
        compute_row(qbase, r)

        @pl.when(r < ROWS_PER_W - RING)
        def _refill():
            pltpu.async_copy(
                x_hbm.at[batch, f0 + r + RING],
                ring_v.at[pl.ds(qbase, NT)],
                sem,
            )

        return lax.rem(q + 1, RING)

    lax.fori_loop(0, ROWS_PER_W, row_body, 0)
    pltpu.sync_copy(out_v, out_hbm.at[batch, pl.ds(f0, ROWS_PER_W)])


def kernel(x, durations):
    return _seg_avg(x, durations.astype(jnp.int32))


# speedup vs baseline: 1.6684x; 1.0350x over previous
import functools

import jax
import jax.numpy as jnp
from jax import lax
from jax.experimental import pallas as pl
from jax.experimental.pallas import tpu as pltpu
from jax.experimental.pallas import tpu_sc as plsc

NB = 16
NF = 80
NT = 8192
NS = 512
MAXD = 15
ROWS_PER_W = 40
RING = 10
DEAD = RING * NT

_mesh = plsc.VectorSubcoreMesh(core_axis_name="c", subcore_axis_name="s")


@functools.partial(
    pl.kernel,
    mesh=_mesh,
    compiler_params=pltpu.CompilerParams(needs_layout_passes=False),
    out_type=jax.ShapeDtypeStruct((NB, NF, NS), jnp.float32),
    scratch_types=[
        pltpu.VMEM((NS,), jnp.int32),
        pltpu.VMEM((NS,), jnp.int32),
        pltpu.VMEM((ROWS_PER_W, NS), jnp.float32),
        pltpu.VMEM((RING * NT + 16,), jnp.float32),
        pltpu.SemaphoreType.DMA,
    ],
)
def _seg_avg(x_hbm, dur_hbm, out_hbm, dur_v, starts_v, out_v, ring_v, sem):
    cid = lax.axis_index("c")
    sid = lax.axis_index("s")
    wid = sid * 2 + cid
    batch = wid // 2
    f0 = (wid % 2) * ROWS_PER_W

    pltpu.sync_copy(dur_hbm.at[batch], dur_v)

    iota = jnp.arange(16, dtype=jnp.int32)
    lane15 = jnp.full((16,), 15, jnp.int32)

    def bounds_body(g, carry_v):
        base = pl.multiple_of(g * 16, 16)
        d = dur_v[pl.ds(base, 16)]
        ends = d
        for k in (1, 2, 4, 8):
            sh = ends.at[jnp.maximum(iota - k, 0)].get(mode="promise_in_bounds")
            ends = ends + jnp.where(iota >= k, sh, 0)
        ends = ends + carry_v
        starts_v[pl.ds(base, 16)] = ends - d
        return ends.at[lane15].get(mode="promise_in_bounds")

    lax.fori_loop(0, NS // 16, bounds_body, jnp.zeros((16,), jnp.int32))

    zeros = jnp.zeros((16,), jnp.float32)
    ring_v[pl.ds(DEAD, 16)] = zeros

    NG = 4

    def compute_row(qbase, r):
        def grp_body(g, _2):
            base = pl.multiple_of(g * (16 * NG), 16)
            s = [starts_v[pl.ds(base + 16 * j, 16)] + qbase for j in range(NG)]
            l = [dur_v[pl.ds(base + 16 * j, 16)] for j in range(NG)]
            acc = [zeros] * NG
            cnt = [zeros] * NG
            for d in range(MAXD):
                idx = [jnp.where(l[j] > d, s[j] + d, DEAD) for j in range(NG)]
                v = [plsc.load_gather(ring_v, [idx[j]]) for j in range(NG)]
                for j in range(NG):
                    acc[j] = acc[j] + v[j]
                    cnt[j] = cnt[j] + jnp.where(v[j] == 0.0, 0.0, 1.0)
            for j in range(NG):
                out_v[r, pl.ds(base + 16 * j, 16)] = (
                    acc[j] / jnp.maximum(cnt[j], 1.0)
                )
            return 0

        lax.fori_loop(0, NS // (16 * NG), grp_body, 0)

    for q in range(RING):
        pltpu.async_copy(
            x_hbm.at[batch, f0 + q], ring_v.at[pl.ds(q * NT, NT)], sem
        )

    def row_body(r, q):
        qbase = pl.multiple_of(q * NT, 16)
        pltpu.make_async_copy(
            x_hbm.at[batch, f0], ring_v.at[pl.ds(qbase, NT)], sem
        ).wait()
        compute_row(qbase, r)

        @pl.when(r < ROWS_PER_W - RING)
        def _refill():
            pltpu.async_copy(
                x_hbm.at[batch, f0 + r + RING],
                ring_v.at[pl.ds(qbase, NT)],
                sem,
            )

        return lax.rem(q + 1, RING)

    lax.fori_loop(0, ROWS_PER_W, row_body, 0)
    pltpu.sync_copy(out_v, out_hbm.at[batch, pl.ds(f0, ROWS_PER_W)])


def kernel(x, durations):
    return _seg_avg(x, durations.astype(jnp.int32))


# speedup vs baseline: 1.7194x; 1.0306x over previous
import functools

import jax
import jax.numpy as jnp
from jax import lax
from jax.experimental import pallas as pl
from jax.experimental.pallas import tpu as pltpu
from jax.experimental.pallas import tpu_sc as plsc

NB = 16
NF = 80
NT = 8192
NS = 512
MAXD = 15
ROWS_PER_W = 40
RING = 10
DEAD = RING * NT

_mesh = plsc.VectorSubcoreMesh(core_axis_name="c", subcore_axis_name="s")


@functools.partial(
    pl.kernel,
    mesh=_mesh,
    compiler_params=pltpu.CompilerParams(needs_layout_passes=False),
    out_type=jax.ShapeDtypeStruct((NB, NF, NS), jnp.float32),
    scratch_types=[
        pltpu.VMEM((NS,), jnp.int32),
        pltpu.VMEM((NS,), jnp.int32),
        pltpu.VMEM((ROWS_PER_W, NS), jnp.float32),
        pltpu.VMEM((RING * NT + 16,), jnp.float32),
        pltpu.SemaphoreType.DMA,
    ],
)
def _seg_avg(x_hbm, dur_hbm, out_hbm, dur_v, starts_v, out_v, ring_v, sem):
    cid = lax.axis_index("c")
    sid = lax.axis_index("s")
    wid = sid * 2 + cid
    batch = wid // 2
    f0 = (wid % 2) * ROWS_PER_W

    pltpu.sync_copy(dur_hbm.at[batch], dur_v)

    iota = jnp.arange(16, dtype=jnp.int32)
    lane15 = jnp.full((16,), 15, jnp.int32)

    def bounds_body(g, carry_v):
        base = pl.multiple_of(g * 16, 16)
        d = dur_v[pl.ds(base, 16)]
        ends = d
        for k in (1, 2, 4, 8):
            sh = ends.at[jnp.maximum(iota - k, 0)].get(mode="promise_in_bounds")
            ends = ends + jnp.where(iota >= k, sh, 0)
        ends = ends + carry_v
        starts_v[pl.ds(base, 16)] = ends - d
        return ends.at[lane15].get(mode="promise_in_bounds")

    lax.fori_loop(0, NS // 16, bounds_body, jnp.zeros((16,), jnp.int32))

    zeros = jnp.zeros((16,), jnp.float32)
    ring_v[pl.ds(DEAD, 16)] = zeros

    NG = 8

    def compute_row(qbase, r):
        def grp_body(g, _2):
            base = pl.multiple_of(g * (16 * NG), 16)
            s = [starts_v[pl.ds(base + 16 * j, 16)] + qbase for j in range(NG)]
            l = [dur_v[pl.ds(base + 16 * j, 16)] for j in range(NG)]
            acc = [zeros] * NG
            cnt = [zeros] * NG
            for d in range(MAXD):
                idx = [jnp.where(l[j] > d, s[j] + d, DEAD) for j in range(NG)]
                v = [plsc.load_gather(ring_v, [idx[j]]) for j in range(NG)]
                for j in range(NG):
                    acc[j] = acc[j] + v[j]
                    cnt[j] = cnt[j] + jnp.where(v[j] == 0.0, 0.0, 1.0)
            for j in range(NG):
                out_v[r, pl.ds(base + 16 * j, 16)] = (
                    acc[j] / jnp.maximum(cnt[j], 1.0)
                )
            return 0

        lax.fori_loop(0, NS // (16 * NG), grp_body, 0)

    for q in range(RING):
        pltpu.async_copy(
            x_hbm.at[batch, f0 + q], ring_v.at[pl.ds(q * NT, NT)], sem
        )

    def row_body(r, q):
        qbase = pl.multiple_of(q * NT, 16)
        pltpu.make_async_copy(
            x_hbm.at[batch, f0], ring_v.at[pl.ds(qbase, NT)], sem
        ).wait()
        compute_row(qbase, r)

        @pl.when(r < ROWS_PER_W - RING)
        def _refill():
            pltpu.async_copy(
                x_hbm.at[batch, f0 + r + RING],
                ring_v.at[pl.ds(qbase, NT)],
                sem,
            )

        return lax.rem(q + 1, RING)

    lax.fori_loop(0, ROWS_PER_W, row_body, 0)
    pltpu.sync_copy(out_v, out_hbm.at[batch, pl.ds(f0, ROWS_PER_W)])


def kernel(x, durations):
    return _seg_avg(x, durations.astype(jnp.int32))


# speedup vs baseline: 1.7751x; 1.0324x over previous
import functools

import jax
import jax.numpy as jnp
from jax import lax
from jax.experimental import pallas as pl
from jax.experimental.pallas import tpu as pltpu
from jax.experimental.pallas import tpu_sc as plsc

NB = 16
NF = 80
NT = 8192
NS = 512
MAXD = 15
ROWS_PER_W = 40
RING = 10
SLOT = NT + 128

_mesh = plsc.VectorSubcoreMesh(core_axis_name="c", subcore_axis_name="s")


@functools.partial(
    pl.kernel,
    mesh=_mesh,
    compiler_params=pltpu.CompilerParams(needs_layout_passes=False),
    out_type=jax.ShapeDtypeStruct((NB, NF, NS), jnp.float32),
    scratch_types=[
        pltpu.VMEM((NS,), jnp.int32),
        pltpu.VMEM((NS,), jnp.int32),
        pltpu.VMEM((ROWS_PER_W, NS), jnp.float32),
        pltpu.VMEM((RING * SLOT,), jnp.float32),
        pltpu.VMEM((NS * MAXD,), jnp.int32),
        pltpu.SemaphoreType.DMA,
    ],
)
def _seg_avg(x_hbm, dur_hbm, out_hbm, dur_v, starts_v, out_v, ring_v, idx_tab,
             sem):
    cid = lax.axis_index("c")
    sid = lax.axis_index("s")
    wid = sid * 2 + cid
    batch = wid // 2
    f0 = (wid % 2) * ROWS_PER_W

    pltpu.sync_copy(dur_hbm.at[batch], dur_v)

    iota = jnp.arange(16, dtype=jnp.int32)
    lane15 = jnp.full((16,), 15, jnp.int32)

    def bounds_body(g, carry_v):
        base = pl.multiple_of(g * 16, 16)
        d = dur_v[pl.ds(base, 16)]
        ends = d
        for k in (1, 2, 4, 8):
            sh = ends.at[jnp.maximum(iota - k, 0)].get(mode="promise_in_bounds")
            ends = ends + jnp.where(iota >= k, sh, 0)
        ends = ends + carry_v
        starts_v[pl.ds(base, 16)] = ends - d
        return ends.at[lane15].get(mode="promise_in_bounds")

    lax.fori_loop(0, NS // 16, bounds_body, jnp.zeros((16,), jnp.int32))

    zeros = jnp.zeros((16,), jnp.float32)
    for q in range(RING):
        ring_v[pl.ds(q * SLOT + NT, 16)] = zeros

    def tab_body(g, _):
        base = pl.multiple_of(g * 16, 16)
        s = starts_v[pl.ds(base, 16)]
        l = dur_v[pl.ds(base, 16)]
        tb = pl.multiple_of(g * (16 * MAXD), 16)
        for d in range(MAXD):
            idx_tab[pl.ds(tb + d * 16, 16)] = jnp.where(l > d, s + d, NT)
        return 0

    lax.fori_loop(0, NS // 16, tab_body, 0)

    NG = 8
    NFLY = 4

    def compute_row(qbase, r):
        dead = qbase + NT

        def grp_body(g, _2):
            base = pl.multiple_of(g * (16 * NG), 16)
            tb = pl.multiple_of(g * (16 * NG * MAXD), 16)
            s = [
                starts_v[pl.ds(base + 16 * j, 16)] + qbase for j in range(NFLY)
            ]
            l = [dur_v[pl.ds(base + 16 * j, 16)] for j in range(NFLY)]
            acc = [zeros] * NG
            cnt = [zeros] * NG
            for d in range(MAXD):
                idx = [jnp.where(l[j] > d, s[j] + d, dead) for j in range(NFLY)]
                idx += [
                    idx_tab[pl.ds(tb + (j * MAXD + d) * 16, 16)] + qbase
                    for j in range(NFLY, NG)
                ]
                v = [plsc.load_gather(ring_v, [idx[j]]) for j in range(NG)]
                for j in range(NG):
                    acc[j] = acc[j] + v[j]
                    cnt[j] = cnt[j] + jnp.where(v[j] == 0.0, 0.0, 1.0)
            for j in range(NG):
                out_v[r, pl.ds(base + 16 * j, 16)] = (
                    acc[j] / jnp.maximum(cnt[j], 1.0)
                )
            return 0

        lax.fori_loop(0, NS // (16 * NG), grp_body, 0)

    for q in range(RING):
        pltpu.async_copy(
            x_hbm.at[batch, f0 + q], ring_v.at[pl.ds(q * SLOT, NT)], sem
        )

    def row_body(r, q):
        qbase = pl.multiple_of(q * SLOT, 16)
        pltpu.make_async_copy(
            x_hbm.at[batch, f0], ring_v.at[pl.ds(qbase, NT)], sem
        ).wait()
        compute_row(qbase, r)

        @pl.when(r < ROWS_PER_W - RING)
        def _refill():
            pltpu.async_copy(
                x_hbm.at[batch, f0 + r + RING],
                ring_v.at[pl.ds(qbase, NT)],
                sem,
            )

        return lax.rem(q + 1, RING)

    lax.fori_loop(0, ROWS_PER_W, row_body, 0)
    pltpu.sync_copy(out_v, out_hbm.at[batch, pl.ds(f0, ROWS_PER_W)])


def kernel(x, durations):
    return _seg_avg(x, durations.astype(jnp.int32))
